# baseline (speedup 1.0000x reference)
import functools

import jax
import jax.numpy as jnp
from jax import lax
from jax.experimental import pallas as pl
from jax.experimental.pallas import tpu as pltpu
from jax.experimental.pallas import tpu_sc as plsc

_NC, _NS, _L = 2, 16, 16
_NW = _NC * _NS
_CE = 128
_FBIG = 80
_BIGCORE = 1


def _proj_body(h_ref, wa_ref, wb_ref, ha_ref, hb_ref):
    hblk = h_ref[...]
    ha_ref[...] = jnp.dot(hblk, wa_ref[...], preferred_element_type=jnp.float32)
    hb_ref[...] = jnp.dot(hblk, wb_ref[...], preferred_element_type=jnp.float32)


def _score_body(ea_ref, gs_ref, w1c_ref, b1_ref, w2_ref, b2_ref,
                s_ref, m_ref, iz_ref, rm, rz):
    i = pl.program_id(0)
    z = jnp.dot(ea_ref[...], w1c_ref[...], preferred_element_type=jnp.float32)
    z = z + gs_ref[...] + b1_ref[...]
    x = jnp.where(z > 0, z, jnp.exp(jnp.minimum(z, 0.0)) - 1.0)
    s = jnp.sum(x * w2_ref[...], axis=1, keepdims=True) + b2_ref[0, 0]
    s = jnp.where(s >= 0, s, 0.2 * s)
    s_ref[...] = s

    bm = jnp.max(s)
    bsum = jnp.sum(jnp.exp(s - bm))

    @pl.when(i == 0)
    def _():
        rm[...] = jnp.full(rm.shape, bm)
        rz[...] = jnp.full(rz.shape, bsum)

    @pl.when(i > 0)
    def _():
        rm_old = rm[...]
        bm_v = jnp.full(rm.shape, bm)
        nm = jnp.maximum(rm_old, bm_v)
        rz[...] = rz[...] * jnp.exp(rm_old - nm) + jnp.full(rz.shape, bsum) * jnp.exp(bm_v - nm)
        rm[...] = nm

    m_ref[...] = rm[...]
    iz_ref[...] = 1.0 / rz[...]


def _final_body(p_ref, iz_ref, o_ref):
    o_ref[...] = (p_ref[0] + p_ref[1]) * iz_ref[...]


def kernel(h, edge_index, edge_attr, W1, b1, W2, b2):
    N, D = h.shape
    E = edge_index.shape[1]
    ndl = D // _L

    row = edge_index[0]
    col = edge_index[1]
    W1a, W1b, W1c = W1[:D], W1[D:2 * D], W1[2 * D:]

    nch = ((-(-E // (_NW * _CE)) + 7) // 8) * 8
    erows = _NW * nch
    epad = erows * _CE
    padn = epad - E
    row2d = jnp.concatenate([row, jnp.zeros((padn,), jnp.int32)]).reshape(erows, _CE)
    col2d = jnp.concatenate([col, jnp.zeros((padn,), jnp.int32)]).reshape(erows, _CE)

    BR = 1000
    nb = N // BR
    hA, hB = pl.pallas_call(
        _proj_body,
        grid=(nb,),
        in_specs=[pl.BlockSpec((BR, D), lambda i: (i, 0)),
                  pl.BlockSpec((D, D), lambda i: (0, 0)),
                  pl.BlockSpec((D, D), lambda i: (0, 0))],
        out_specs=[pl.BlockSpec((BR, D), lambda i: (i, 0)),
                   pl.BlockSpec((BR, D), lambda i: (i, 0))],
        out_shape=[jax.ShapeDtypeStruct((N, D), jnp.float32),
                   jax.ShapeDtypeStruct((N, D), jnp.float32)],
    )(h, W1a, W1b)

    mesh = plsc.VectorSubcoreMesh(core_axis_name="c", subcore_axis_name="s",
                                  num_cores=_NC, num_subcores=_NS)

    nf, ns = _FBIG, nch * 2 - _FBIG

    @functools.partial(
        pl.kernel,
        out_type=jax.ShapeDtypeStruct((epad, D), jnp.float32),
        mesh=mesh,
        scratch_types=[pltpu.VMEM((nf, _CE), jnp.int32),
                       pltpu.VMEM((nf, _CE), jnp.int32),
                       pltpu.VMEM((_CE, D), jnp.float32),
                       pltpu.VMEM((_CE, D), jnp.float32),
                       pltpu.VMEM((_CE, D), jnp.float32),
                       pltpu.VMEM((_CE, D), jnp.float32),
                       pltpu.VMEM((_CE, D), jnp.float32),
                       pltpu.VMEM((_CE, D), jnp.float32),
                       pltpu.SemaphoreType.DMA,
                       pltpu.SemaphoreType.DMA,
                       pltpu.SemaphoreType.DMA,
                       pltpu.SemaphoreType.DMA,
                       pltpu.SemaphoreType.DMA,
                       pltpu.SemaphoreType.DMA],
    )
    def _sc_gather_sum(ha_hbm, hb_hbm, row_hbm, col_hbm, out_hbm,
                       idxr, idxc, a0, b0, o0, a1, b1s, o1,
                       ga0, gb0, ga1, gb1, w0, w1s):
        cid = lax.axis_index("c")
        sid = lax.axis_index("s")
        abufs, bbufs, obufs = (a0, a1), (b0, b1s), (o0, o1)
        gasems, gbsems, wsems = (ga0, ga1), (gb0, gb1), (w0, w1s)

        def run(nch_c, row_base):
            pltpu.sync_copy(row_hbm.at[pl.ds(row_base, nch_c)],
                            idxr.at[pl.ds(0, nch_c)])
            pltpu.sync_copy(col_hbm.at[pl.ds(row_base, nch_c)],
                            idxc.at[pl.ds(0, nch_c)])

            def step(k, carry):
                cpa = pltpu.async_copy(ha_hbm.at[idxr.at[k]], a0, ga0)
                cpb = pltpu.async_copy(hb_hbm.at[idxc.at[k]], b0, gb0)
                cpa.wait()
                cpb.wait()

                def addrow(r, c2):
                    for dd in range(ndl):
                        sl = pl.ds(dd * _L, _L)
                        o0[r, sl] = a0[r, sl] + b0[r, sl]
                    return c2

                lax.fori_loop(0, _CE, addrow, 0)
                base = (row_base + k) * _CE
                pltpu.sync_copy(o0, out_hbm.at[pl.ds(base, _CE)])
                return carry

            lax.fori_loop(0, nch_c, step, 0)

        @pl.when(cid == _BIGCORE)
        def _():
            run(nf, sid * nf)

        @pl.when(cid != _BIGCORE)
        def _():
            run(ns, _NS * nf + sid * ns)

    gsum = _sc_gather_sum(hA, hB, row2d, col2d)

    EB = 2560
    nbe = E // EB
    b2r = jnp.broadcast_to(b2.reshape(1, 1), (1, D))
    s, mvec, izvec = pl.pallas_call(
        _score_body,
        grid=(nbe,),
        in_specs=[pl.BlockSpec((EB, D), lambda i: (i, 0)),
                  pl.BlockSpec((EB, D), lambda i: (i, 0)),
                  pl.BlockSpec((D, D), lambda i: (0, 0)),
                  pl.BlockSpec((1, D), lambda i: (0, 0)),
                  pl.BlockSpec((1, D), lambda i: (0, 0)),
                  pl.BlockSpec((1, D), lambda i: (0, 0))],
        out_specs=[pl.BlockSpec((EB, 1), lambda i: (i, 0)),
                   pl.BlockSpec((1, D), lambda i: (0, 0)),
                   pl.BlockSpec((1, D), lambda i: (0, 0))],
        out_shape=[jax.ShapeDtypeStruct((E, 1), jnp.float32),
                   jax.ShapeDtypeStruct((1, D), jnp.float32),
                   jax.ShapeDtypeStruct((1, D), jnp.float32)],
        scratch_shapes=[pltpu.VMEM((1, D), jnp.float32),
                        pltpu.VMEM((1, D), jnp.float32)],
    )(edge_attr, gsum, W1c, b1.reshape(1, D), W2.reshape(1, D), b2r)

    s1d = jnp.concatenate([s.reshape(E), jnp.full((padn,), -1e30, jnp.float32)])
    row1d = jnp.concatenate([row, jnp.zeros((padn,), jnp.int32)])
    m16 = lax.slice(mvec, (0, 0), (1, _L)).reshape(_L)

    drain = 128
    npad = ((N + drain * _NS - 1) // (drain * _NS)) * (drain * _NS)
    rps = npad // _NS
    ndrain = rps // drain

    @functools.partial(
        pl.kernel,
        out_type=jax.ShapeDtypeStruct((_NC, npad, D), jnp.float32),
        mesh=mesh,
        scratch_types=[pltpu.VMEM((nf, _CE), jnp.int32),
                       pltpu.VMEM((_CE,), jnp.int32),
                       pltpu.VMEM((_CE,), jnp.int32),
                       pltpu.VMEM((_CE,), jnp.float32),
                       pltpu.VMEM((_CE,), jnp.float32),
                       pltpu.VMEM((_CE, D), jnp.float32),
                       pltpu.VMEM((_CE, D), jnp.float32),
                       pltpu.VMEM((_L,), jnp.float32),
                       pltpu.VMEM_SHARED((npad, D), jnp.float32),
                       pltpu.SemaphoreType.DMA,
                       pltpu.SemaphoreType.DMA,
                       pltpu.SemaphoreType.DMA,
                       pltpu.SemaphoreType.DMA],
    )
    def _sc_scatter(h_hbm, col_hbm, row_hbm, s_hbm, m_hbm, out_hbm,
                    cidx, r0i, r1i, s0b, s1b, g0, g1, m_v, acc,
                    gs0, gs1, is0, is1):
        cid = lax.axis_index("c")
        sid = lax.axis_index("s")

        def zrow(r, c2):
            for dd in range(ndl):
                g0[r, pl.ds(dd * _L, _L)] = jnp.zeros((_L,), jnp.float32)
            return c2

        lax.fori_loop(0, _CE, zrow, 0)
        for j in range(ndrain):
            pltpu.sync_copy(g0, acc.at[pl.ds(sid * rps + j * drain, drain)])

        gbufs = (g0, g1)
        ribufs, sbufs = (r0i, r1i), (s0b, s1b)
        gsems, isems = (gs0, gs1), (is0, is1)

        def run(nch_c, row_base):
            ebase = row_base * _CE
            pltpu.sync_copy(col_hbm.at[pl.ds(row_base, nch_c)],
                            cidx.at[pl.ds(0, nch_c)])
            pltpu.sync_copy(row_hbm.at[pl.ds(ebase, _CE)], r0i)
            pltpu.sync_copy(row_hbm.at[pl.ds(ebase + _CE, _CE)], r1i)
            pltpu.sync_copy(s_hbm.at[pl.ds(ebase, _CE)], s0b)
            pltpu.sync_copy(s_hbm.at[pl.ds(ebase + _CE, _CE)], s1b)
            pltpu.sync_copy(m_hbm, m_v)
            plsc.subcore_barrier()
            m16v = m_v[...]

            def gissue(k, slot):
                pltpu.async_copy(h_hbm.at[cidx.at[k]], gbufs[slot], gsems[slot])

            gissue(0, 0)
            gissue(1, 1)

            def step(t, carry):
                for slot in range(2):
                    k = 2 * t + slot
                    pltpu.make_async_copy(h_hbm.at[cidx.at[k]], gbufs[slot],
                                          gsems[slot]).wait()

                    @pl.when(t > 0)
                    def _(slot=slot, k=k):
                        off = ebase + k * _CE
                        pltpu.make_async_copy(row_hbm.at[pl.ds(off, _CE)],
                                              ribufs[slot], isems[slot]).wait()
                        pltpu.make_async_copy(s_hbm.at[pl.ds(off, _CE)],
                                              sbufs[slot], isems[slot]).wait()

                    gb, sb = gbufs[slot], sbufs[slot]

                    def scale(g, c2, gb=gb, sb=sb):
                        sv = sb[pl.ds(g * _L, _L)]
                        w16 = jnp.exp(sv - m16v)
                        for l in range(_L):
                            e = g * _L + l
                            wsc = w16[l]
                            for dd in range(ndl):
                                sl = pl.ds(dd * _L, _L)
                                gb[e, sl] = gb[e, sl] * wsc
                        return c2

                    lax.fori_loop(0, _CE // _L, scale, 0)
                    pltpu.sync_copy(gb, acc.at[ribufs[slot]], add=True)

                    @pl.when(k + 2 < nch_c)
                    def _(k=k, slot=slot):
                        off2 = ebase + (k + 2) * _CE
                        pltpu.async_copy(row_hbm.at[pl.ds(off2, _CE)],
                                         ribufs[slot], isems[slot])
                        pltpu.async_copy(s_hbm.at[pl.ds(off2, _CE)],
                                         sbufs[slot], isems[slot])
                        gissue(k + 2, slot)
                return carry

            lax.fori_loop(0, nch_c // 2, step, 0)

        @pl.when(cid == _BIGCORE)
        def _():
            run(nf, sid * nf)

        @pl.when(cid != _BIGCORE)
        def _():
            run(ns, _NS * nf + sid * ns)

        plsc.subcore_barrier()

        for j in range(ndrain):
            r0 = sid * rps + j * drain
            pltpu.sync_copy(acc.at[pl.ds(r0, drain)], g0)
            pltpu.sync_copy(g0, out_hbm.at[cid, pl.ds(r0, drain)])

    part = _sc_scatter(h, col2d, row1d, s1d, m16)

    out = pl.pallas_call(
        _final_body,
        grid=(nb,),
        in_specs=[pl.BlockSpec((_NC, BR, D), lambda i: (0, i, 0)),
                  pl.BlockSpec((1, D), lambda i: (0, 0))],
        out_specs=pl.BlockSpec((BR, D), lambda i: (i, 0)),
        out_shape=jax.ShapeDtypeStruct((N, D), jnp.float32),
    )(part, izvec)
    return out


# speedup vs baseline: 1.1159x; 1.1159x over previous
import functools

import jax
import jax.numpy as jnp
from jax import lax
from jax.experimental import pallas as pl
from jax.experimental.pallas import tpu as pltpu
from jax.experimental.pallas import tpu_sc as plsc

_NC, _NS, _L = 2, 16, 16
_NW = _NC * _NS
_CE = 128
_FBIG = 80
_BIGCORE = 1


def _proj_body(h_ref, wa_ref, wb_ref, ha_ref, hb_ref):
    hblk = h_ref[...]
    ha_ref[...] = jnp.dot(hblk, wa_ref[...], preferred_element_type=jnp.float32)
    hb_ref[...] = jnp.dot(hblk, wb_ref[...], preferred_element_type=jnp.float32)


def _score_body(ea_ref, gs_ref, w1c_ref, b1_ref, w2_ref, b2_ref,
                s_ref, m_ref, iz_ref, rm, rz):
    i = pl.program_id(0)
    z = jnp.dot(ea_ref[...], w1c_ref[...], preferred_element_type=jnp.float32)
    z = z + gs_ref[...] + b1_ref[...]
    x = jnp.where(z > 0, z, jnp.exp(jnp.minimum(z, 0.0)) - 1.0)
    s = jnp.sum(x * w2_ref[...], axis=1, keepdims=True) + b2_ref[0, 0]
    s = jnp.where(s >= 0, s, 0.2 * s)
    s_ref[...] = s

    bm = jnp.max(s)
    bsum = jnp.sum(jnp.exp(s - bm))

    @pl.when(i == 0)
    def _():
        rm[...] = jnp.full(rm.shape, bm)
        rz[...] = jnp.full(rz.shape, bsum)

    @pl.when(i > 0)
    def _():
        rm_old = rm[...]
        bm_v = jnp.full(rm.shape, bm)
        nm = jnp.maximum(rm_old, bm_v)
        rz[...] = rz[...] * jnp.exp(rm_old - nm) + jnp.full(rz.shape, bsum) * jnp.exp(bm_v - nm)
        rm[...] = nm

    m_ref[...] = rm[...]
    iz_ref[...] = 1.0 / rz[...]


def _final_body(p_ref, iz_ref, o_ref):
    o_ref[...] = (p_ref[0] + p_ref[1]) * iz_ref[...]


def kernel(h, edge_index, edge_attr, W1, b1, W2, b2):
    N, D = h.shape
    E = edge_index.shape[1]
    ndl = D // _L

    row = edge_index[0]
    col = edge_index[1]
    W1a, W1b, W1c = W1[:D], W1[D:2 * D], W1[2 * D:]

    nch = ((-(-E // (_NW * _CE)) + 7) // 8) * 8
    erows = _NW * nch
    epad = erows * _CE
    padn = epad - E
    row2d = jnp.concatenate([row, jnp.zeros((padn,), jnp.int32)]).reshape(erows, _CE)
    col2d = jnp.concatenate([col, jnp.zeros((padn,), jnp.int32)]).reshape(erows, _CE)

    BR = 1000
    nb = N // BR
    hA, hB = pl.pallas_call(
        _proj_body,
        grid=(nb,),
        in_specs=[pl.BlockSpec((BR, D), lambda i: (i, 0)),
                  pl.BlockSpec((D, D), lambda i: (0, 0)),
                  pl.BlockSpec((D, D), lambda i: (0, 0))],
        out_specs=[pl.BlockSpec((BR, D), lambda i: (i, 0)),
                   pl.BlockSpec((BR, D), lambda i: (i, 0))],
        out_shape=[jax.ShapeDtypeStruct((N, D), jnp.float32),
                   jax.ShapeDtypeStruct((N, D), jnp.float32)],
    )(h, W1a, W1b)

    mesh = plsc.VectorSubcoreMesh(core_axis_name="c", subcore_axis_name="s",
                                  num_cores=_NC, num_subcores=_NS)

    nf, ns = _FBIG, nch * 2 - _FBIG

    @functools.partial(
        pl.kernel,
        out_type=jax.ShapeDtypeStruct((epad, D), jnp.float32),
        mesh=mesh,
        scratch_types=[pltpu.VMEM((nf, _CE), jnp.int32),
                       pltpu.VMEM((nf, _CE), jnp.int32),
                       pltpu.VMEM((_CE, D), jnp.float32),
                       pltpu.VMEM((_CE, D), jnp.float32),
                       pltpu.VMEM((_CE, D), jnp.float32),
                       pltpu.VMEM((_CE, D), jnp.float32),
                       pltpu.VMEM((_CE, D), jnp.float32),
                       pltpu.VMEM((_CE, D), jnp.float32),
                       pltpu.SemaphoreType.DMA,
                       pltpu.SemaphoreType.DMA,
                       pltpu.SemaphoreType.DMA,
                       pltpu.SemaphoreType.DMA,
                       pltpu.SemaphoreType.DMA,
                       pltpu.SemaphoreType.DMA],
    )
    def _sc_gather_sum(ha_hbm, hb_hbm, row_hbm, col_hbm, out_hbm,
                       idxr, idxc, a0, b0, o0, a1, b1s, o1,
                       ga0, gb0, ga1, gb1, w0, w1s):
        cid = lax.axis_index("c")
        sid = lax.axis_index("s")
        abufs, bbufs, obufs = (a0, a1), (b0, b1s), (o0, o1)
        gasems, gbsems, wsems = (ga0, ga1), (gb0, gb1), (w0, w1s)

        def run(nch_c, row_base):
            pltpu.sync_copy(row_hbm.at[pl.ds(row_base, nch_c)],
                            idxr.at[pl.ds(0, nch_c)])
            pltpu.sync_copy(col_hbm.at[pl.ds(row_base, nch_c)],
                            idxc.at[pl.ds(0, nch_c)])

            def gissue(k, slot):
                pltpu.async_copy(ha_hbm.at[idxr.at[k]], abufs[slot], gasems[slot])
                pltpu.async_copy(hb_hbm.at[idxc.at[k]], bbufs[slot], gbsems[slot])

            gissue(0, 0)
            gissue(1, 1)

            def step(t, carry):
                for slot in range(2):
                    k = 2 * t + slot
                    pltpu.make_async_copy(ha_hbm.at[idxr.at[k]], abufs[slot],
                                          gasems[slot]).wait()
                    pltpu.make_async_copy(hb_hbm.at[idxc.at[k]], bbufs[slot],
                                          gbsems[slot]).wait()

                    @pl.when(t > 0)
                    def _(slot=slot):
                        pltpu.make_async_copy(obufs[slot], out_hbm.at[pl.ds(0, _CE)],
                                              wsems[slot]).wait()

                    a, b, o = abufs[slot], bbufs[slot], obufs[slot]

                    def addrow(r, c2, a=a, b=b, o=o):
                        for dd in range(ndl):
                            sl = pl.ds(dd * _L, _L)
                            o[r, sl] = a[r, sl] + b[r, sl]
                        return c2

                    lax.fori_loop(0, _CE, addrow, 0)

                    @pl.when(k + 2 < nch_c)
                    def _(k=k, slot=slot):
                        gissue(k + 2, slot)

                    base = (row_base + k) * _CE
                    pltpu.async_copy(o, out_hbm.at[pl.ds(base, _CE)], wsems[slot])
                return carry

            lax.fori_loop(0, nch_c // 2, step, 0)
            pltpu.make_async_copy(o0, out_hbm.at[pl.ds(0, _CE)], w0).wait()
            pltpu.make_async_copy(o1, out_hbm.at[pl.ds(0, _CE)], w1s).wait()

        @pl.when(cid == _BIGCORE)
        def _():
            run(nf, sid * nf)

        @pl.when(cid != _BIGCORE)
        def _():
            run(ns, _NS * nf + sid * ns)

    gsum = _sc_gather_sum(hA, hB, row2d, col2d)

    EB = 2560
    nbe = E // EB
    b2r = jnp.broadcast_to(b2.reshape(1, 1), (1, D))
    s, mvec, izvec = pl.pallas_call(
        _score_body,
        grid=(nbe,),
        in_specs=[pl.BlockSpec((EB, D), lambda i: (i, 0)),
                  pl.BlockSpec((EB, D), lambda i: (i, 0)),
                  pl.BlockSpec((D, D), lambda i: (0, 0)),
                  pl.BlockSpec((1, D), lambda i: (0, 0)),
                  pl.BlockSpec((1, D), lambda i: (0, 0)),
                  pl.BlockSpec((1, D), lambda i: (0, 0))],
        out_specs=[pl.BlockSpec((EB, 1), lambda i: (i, 0)),
                   pl.BlockSpec((1, D), lambda i: (0, 0)),
                   pl.BlockSpec((1, D), lambda i: (0, 0))],
        out_shape=[jax.ShapeDtypeStruct((E, 1), jnp.float32),
                   jax.ShapeDtypeStruct((1, D), jnp.float32),
                   jax.ShapeDtypeStruct((1, D), jnp.float32)],
        scratch_shapes=[pltpu.VMEM((1, D), jnp.float32),
                        pltpu.VMEM((1, D), jnp.float32)],
    )(edge_attr, gsum, W1c, b1.reshape(1, D), W2.reshape(1, D), b2r)

    s1d = jnp.concatenate([s.reshape(E), jnp.full((padn,), -1e30, jnp.float32)])
    row1d = jnp.concatenate([row, jnp.zeros((padn,), jnp.int32)])
    m16 = lax.slice(mvec, (0, 0), (1, _L)).reshape(_L)

    drain = 128
    npad = ((N + drain * _NS - 1) // (drain * _NS)) * (drain * _NS)
    rps = npad // _NS
    ndrain = rps // drain

    @functools.partial(
        pl.kernel,
        out_type=jax.ShapeDtypeStruct((_NC, npad, D), jnp.float32),
        mesh=mesh,
        scratch_types=[pltpu.VMEM((nf, _CE), jnp.int32),
                       pltpu.VMEM((_CE,), jnp.int32),
                       pltpu.VMEM((_CE,), jnp.int32),
                       pltpu.VMEM((_CE,), jnp.float32),
                       pltpu.VMEM((_CE,), jnp.float32),
                       pltpu.VMEM((_CE, D), jnp.float32),
                       pltpu.VMEM((_CE, D), jnp.float32),
                       pltpu.VMEM((_L,), jnp.float32),
                       pltpu.VMEM_SHARED((npad, D), jnp.float32),
                       pltpu.SemaphoreType.DMA,
                       pltpu.SemaphoreType.DMA,
                       pltpu.SemaphoreType.DMA,
                       pltpu.SemaphoreType.DMA],
    )
    def _sc_scatter(h_hbm, col_hbm, row_hbm, s_hbm, m_hbm, out_hbm,
                    cidx, r0i, r1i, s0b, s1b, g0, g1, m_v, acc,
                    gs0, gs1, is0, is1):
        cid = lax.axis_index("c")
        sid = lax.axis_index("s")

        def zrow(r, c2):
            for dd in range(ndl):
                g0[r, pl.ds(dd * _L, _L)] = jnp.zeros((_L,), jnp.float32)
            return c2

        lax.fori_loop(0, _CE, zrow, 0)
        for j in range(ndrain):
            pltpu.sync_copy(g0, acc.at[pl.ds(sid * rps + j * drain, drain)])

        gbufs = (g0, g1)
        ribufs, sbufs = (r0i, r1i), (s0b, s1b)
        gsems, isems = (gs0, gs1), (is0, is1)

        def run(nch_c, row_base):
            ebase = row_base * _CE
            pltpu.sync_copy(col_hbm.at[pl.ds(row_base, nch_c)],
                            cidx.at[pl.ds(0, nch_c)])
            pltpu.sync_copy(row_hbm.at[pl.ds(ebase, _CE)], r0i)
            pltpu.sync_copy(row_hbm.at[pl.ds(ebase + _CE, _CE)], r1i)
            pltpu.sync_copy(s_hbm.at[pl.ds(ebase, _CE)], s0b)
            pltpu.sync_copy(s_hbm.at[pl.ds(ebase + _CE, _CE)], s1b)
            pltpu.sync_copy(m_hbm, m_v)
            plsc.subcore_barrier()
            m16v = m_v[...]

            def gissue(k, slot):
                pltpu.async_copy(h_hbm.at[cidx.at[k]], gbufs[slot], gsems[slot])

            gissue(0, 0)
            gissue(1, 1)

            def step(t, carry):
                for slot in range(2):
                    k = 2 * t + slot
                    pltpu.make_async_copy(h_hbm.at[cidx.at[k]], gbufs[slot],
                                          gsems[slot]).wait()

                    @pl.when(t > 0)
                    def _(slot=slot, k=k):
                        off = ebase + k * _CE
                        pltpu.make_async_copy(row_hbm.at[pl.ds(off, _CE)],
                                              ribufs[slot], isems[slot]).wait()
                        pltpu.make_async_copy(s_hbm.at[pl.ds(off, _CE)],
                                              sbufs[slot], isems[slot]).wait()

                    gb, sb = gbufs[slot], sbufs[slot]

                    def scale(g, c2, gb=gb, sb=sb):
                        sv = sb[pl.ds(g * _L, _L)]
                        w16 = jnp.exp(sv - m16v)
                        for l in range(_L):
                            e = g * _L + l
                            wsc = w16[l]
                            for dd in range(ndl):
                                sl = pl.ds(dd * _L, _L)
                                gb[e, sl] = gb[e, sl] * wsc
                        return c2

                    lax.fori_loop(0, _CE // _L, scale, 0)
                    pltpu.sync_copy(gb, acc.at[ribufs[slot]], add=True)

                    @pl.when(k + 2 < nch_c)
                    def _(k=k, slot=slot):
                        off2 = ebase + (k + 2) * _CE
                        pltpu.async_copy(row_hbm.at[pl.ds(off2, _CE)],
                                         ribufs[slot], isems[slot])
                        pltpu.async_copy(s_hbm.at[pl.ds(off2, _CE)],
                                         sbufs[slot], isems[slot])
                        gissue(k + 2, slot)
                return carry

            lax.fori_loop(0, nch_c // 2, step, 0)

        @pl.when(cid == _BIGCORE)
        def _():
            run(nf, sid * nf)

        @pl.when(cid != _BIGCORE)
        def _():
            run(ns, _NS * nf + sid * ns)

        plsc.subcore_barrier()

        for j in range(ndrain):
            r0 = sid * rps + j * drain
            pltpu.sync_copy(acc.at[pl.ds(r0, drain)], g0)
            pltpu.sync_copy(g0, out_hbm.at[cid, pl.ds(r0, drain)])

    part = _sc_scatter(h, col2d, row1d, s1d, m16)

    out = pl.pallas_call(
        _final_body,
        grid=(nb,),
        in_specs=[pl.BlockSpec((_NC, BR, D), lambda i: (0, i, 0)),
                  pl.BlockSpec((1, D), lambda i: (0, 0))],
        out_specs=pl.BlockSpec((BR, D), lambda i: (i, 0)),
        out_shape=jax.ShapeDtypeStruct((N, D), jnp.float32),
    )(part, izvec)
    return out


# speedup vs baseline: 1.1824x; 1.0596x over previous
import functools

import jax
import jax.numpy as jnp
from jax import lax
from jax.experimental import pallas as pl
from jax.experimental.pallas import tpu as pltpu
from jax.experimental.pallas import tpu_sc as plsc

_NC, _NS, _L = 2, 16, 16
_NW = _NC * _NS
_CE = 128
_FBIG = 80
_BIGCORE = 1


def _proj_body(h_ref, wa_ref, wb_ref, ha_ref, hb_ref):
    hblk = h_ref[...]
    ha_ref[...] = jnp.dot(hblk, wa_ref[...], preferred_element_type=jnp.float32)
    hb_ref[...] = jnp.dot(hblk, wb_ref[...], preferred_element_type=jnp.float32)


def _score_body(ea_ref, gs_ref, w1c_ref, b1_ref, w2_ref, b2_ref,
                s_ref, m_ref, iz_ref, rm, rz):
    i = pl.program_id(0)
    z = jnp.dot(ea_ref[...], w1c_ref[...], preferred_element_type=jnp.float32)
    z = z + gs_ref[...] + b1_ref[...]
    x = jnp.where(z > 0, z, jnp.exp(jnp.minimum(z, 0.0)) - 1.0)
    s = jnp.sum(x * w2_ref[...], axis=1, keepdims=True) + b2_ref[0, 0]
    s = jnp.where(s >= 0, s, 0.2 * s)
    s_ref[...] = s

    bm = jnp.max(s)
    bsum = jnp.sum(jnp.exp(s - bm))

    @pl.when(i == 0)
    def _():
        rm[...] = jnp.full(rm.shape, bm)
        rz[...] = jnp.full(rz.shape, bsum)

    @pl.when(i > 0)
    def _():
        rm_old = rm[...]
        bm_v = jnp.full(rm.shape, bm)
        nm = jnp.maximum(rm_old, bm_v)
        rz[...] = rz[...] * jnp.exp(rm_old - nm) + jnp.full(rz.shape, bsum) * jnp.exp(bm_v - nm)
        rm[...] = nm

    m_ref[...] = rm[...]
    iz_ref[...] = 1.0 / rz[...]


def _final_body(p_ref, iz_ref, o_ref):
    o_ref[...] = (p_ref[0] + p_ref[1]) * iz_ref[...]


def kernel(h, edge_index, edge_attr, W1, b1, W2, b2):
    N, D = h.shape
    E = edge_index.shape[1]
    ndl = D // _L

    row = edge_index[0]
    col = edge_index[1]
    W1a, W1b, W1c = W1[:D], W1[D:2 * D], W1[2 * D:]

    nch = ((-(-E // (_NW * _CE)) + 7) // 8) * 8
    erows = _NW * nch
    epad = erows * _CE
    padn = epad - E
    row2d = jnp.concatenate([row, jnp.zeros((padn,), jnp.int32)]).reshape(erows, _CE)
    col2d = jnp.concatenate([col, jnp.zeros((padn,), jnp.int32)]).reshape(erows, _CE)

    BR = 1000
    nb = N // BR
    hA, hB = pl.pallas_call(
        _proj_body,
        grid=(nb,),
        in_specs=[pl.BlockSpec((BR, D), lambda i: (i, 0)),
                  pl.BlockSpec((D, D), lambda i: (0, 0)),
                  pl.BlockSpec((D, D), lambda i: (0, 0))],
        out_specs=[pl.BlockSpec((BR, D), lambda i: (i, 0)),
                   pl.BlockSpec((BR, D), lambda i: (i, 0))],
        out_shape=[jax.ShapeDtypeStruct((N, D), jnp.float32),
                   jax.ShapeDtypeStruct((N, D), jnp.float32)],
    )(h, W1a, W1b)

    mesh = plsc.VectorSubcoreMesh(core_axis_name="c", subcore_axis_name="s",
                                  num_cores=_NC, num_subcores=_NS)

    nf, ns = _FBIG, nch * 2 - _FBIG

    @functools.partial(
        pl.kernel,
        out_type=jax.ShapeDtypeStruct((epad, D), jnp.float32),
        mesh=mesh,
        scratch_types=[pltpu.VMEM((nf, _CE), jnp.int32),
                       pltpu.VMEM((nf, _CE), jnp.int32),
                       pltpu.VMEM((_CE, D), jnp.float32),
                       pltpu.VMEM((_CE, D), jnp.float32),
                       pltpu.VMEM((_CE, D), jnp.float32),
                       pltpu.VMEM((_CE, D), jnp.float32),
                       pltpu.VMEM((_CE, D), jnp.float32),
                       pltpu.VMEM((_CE, D), jnp.float32),
                       pltpu.SemaphoreType.DMA,
                       pltpu.SemaphoreType.DMA,
                       pltpu.SemaphoreType.DMA,
                       pltpu.SemaphoreType.DMA,
                       pltpu.SemaphoreType.DMA,
                       pltpu.SemaphoreType.DMA],
    )
    def _sc_gather_sum(ha_hbm, hb_hbm, row_hbm, col_hbm, out_hbm,
                       idxr, idxc, a0, b0, o0, a1, b1s, o1,
                       ga0, gb0, ga1, gb1, w0, w1s):
        cid = lax.axis_index("c")
        sid = lax.axis_index("s")
        abufs, bbufs, obufs = (a0, a1), (b0, b1s), (o0, o1)
        gasems, gbsems, wsems = (ga0, ga1), (gb0, gb1), (w0, w1s)

        def run(nch_c, row_base):
            pltpu.sync_copy(row_hbm.at[pl.ds(row_base, nch_c)],
                            idxr.at[pl.ds(0, nch_c)])
            pltpu.sync_copy(col_hbm.at[pl.ds(row_base, nch_c)],
                            idxc.at[pl.ds(0, nch_c)])

            def gissue(k, slot):
                pltpu.async_copy(ha_hbm.at[idxr.at[k]], abufs[slot], gasems[slot])
                pltpu.async_copy(hb_hbm.at[idxc.at[k]], bbufs[slot], gbsems[slot])

            gissue(0, 0)
            gissue(1, 1)

            def step(t, carry):
                for slot in range(2):
                    k = 2 * t + slot
                    pltpu.make_async_copy(ha_hbm.at[idxr.at[k]], abufs[slot],
                                          gasems[slot]).wait()
                    pltpu.make_async_copy(hb_hbm.at[idxc.at[k]], bbufs[slot],
                                          gbsems[slot]).wait()

                    @pl.when(t > 0)
                    def _(slot=slot):
                        pltpu.make_async_copy(obufs[slot], out_hbm.at[pl.ds(0, _CE)],
                                              wsems[slot]).wait()

                    a, b, o = abufs[slot], bbufs[slot], obufs[slot]

                    def addrow(r, c2, a=a, b=b, o=o):
                        for dd in range(ndl):
                            sl = pl.ds(dd * _L, _L)
                            o[r, sl] = a[r, sl] + b[r, sl]
                        return c2

                    lax.fori_loop(0, _CE, addrow, 0)

                    @pl.when(k + 2 < nch_c)
                    def _(k=k, slot=slot):
                        gissue(k + 2, slot)

                    base = (row_base + k) * _CE
                    pltpu.async_copy(o, out_hbm.at[pl.ds(base, _CE)], wsems[slot])
                return carry

            lax.fori_loop(0, nch_c // 2, step, 0)
            pltpu.make_async_copy(o0, out_hbm.at[pl.ds(0, _CE)], w0).wait()
            pltpu.make_async_copy(o1, out_hbm.at[pl.ds(0, _CE)], w1s).wait()

        @pl.when(cid == _BIGCORE)
        def _():
            run(nf, sid * nf)

        @pl.when(cid != _BIGCORE)
        def _():
            run(ns, _NS * nf + sid * ns)

    gsum = _sc_gather_sum(hA, hB, row2d, col2d)

    EB = 6400
    nbe = E // EB
    b2r = jnp.broadcast_to(b2.reshape(1, 1), (1, D))
    s, mvec, izvec = pl.pallas_call(
        _score_body,
        grid=(nbe,),
        in_specs=[pl.BlockSpec((EB, D), lambda i: (i, 0)),
                  pl.BlockSpec((EB, D), lambda i: (i, 0)),
                  pl.BlockSpec((D, D), lambda i: (0, 0)),
                  pl.BlockSpec((1, D), lambda i: (0, 0)),
                  pl.BlockSpec((1, D), lambda i: (0, 0)),
                  pl.BlockSpec((1, D), lambda i: (0, 0))],
        out_specs=[pl.BlockSpec((EB, 1), lambda i: (i, 0)),
                   pl.BlockSpec((1, D), lambda i: (0, 0)),
                   pl.BlockSpec((1, D), lambda i: (0, 0))],
        out_shape=[jax.ShapeDtypeStruct((E, 1), jnp.float32),
                   jax.ShapeDtypeStruct((1, D), jnp.float32),
                   jax.ShapeDtypeStruct((1, D), jnp.float32)],
        scratch_shapes=[pltpu.VMEM((1, D), jnp.float32),
                        pltpu.VMEM((1, D), jnp.float32)],
    )(edge_attr, gsum, W1c, b1.reshape(1, D), W2.reshape(1, D), b2r)

    s1d = jnp.concatenate([s.reshape(E), jnp.full((padn,), -1e30, jnp.float32)])
    row1d = jnp.concatenate([row, jnp.zeros((padn,), jnp.int32)])
    m16 = lax.slice(mvec, (0, 0), (1, _L)).reshape(_L)

    drain = 128
    npad = ((N + drain * _NS - 1) // (drain * _NS)) * (drain * _NS)
    rps = npad // _NS
    ndrain = rps // drain

    @functools.partial(
        pl.kernel,
        out_type=jax.ShapeDtypeStruct((_NC, npad, D), jnp.float32),
        mesh=mesh,
        scratch_types=[pltpu.VMEM((nf, _CE), jnp.int32),
                       pltpu.VMEM((_CE,), jnp.int32),
                       pltpu.VMEM((_CE,), jnp.int32),
                       pltpu.VMEM((_CE,), jnp.float32),
                       pltpu.VMEM((_CE,), jnp.float32),
                       pltpu.VMEM((_CE, D), jnp.float32),
                       pltpu.VMEM((_CE, D), jnp.float32),
                       pltpu.VMEM((_L,), jnp.float32),
                       pltpu.VMEM_SHARED((npad, D), jnp.float32),
                       pltpu.SemaphoreType.DMA,
                       pltpu.SemaphoreType.DMA,
                       pltpu.SemaphoreType.DMA,
                       pltpu.SemaphoreType.DMA],
    )
    def _sc_scatter(h_hbm, col_hbm, row_hbm, s_hbm, m_hbm, out_hbm,
                    cidx, r0i, r1i, s0b, s1b, g0, g1, m_v, acc,
                    gs0, gs1, is0, is1):
        cid = lax.axis_index("c")
        sid = lax.axis_index("s")

        def zrow(r, c2):
            for dd in range(ndl):
                g0[r, pl.ds(dd * _L, _L)] = jnp.zeros((_L,), jnp.float32)
            return c2

        lax.fori_loop(0, _CE, zrow, 0)
        for j in range(ndrain):
            pltpu.sync_copy(g0, acc.at[pl.ds(sid * rps + j * drain, drain)])

        gbufs = (g0, g1)
        ribufs, sbufs = (r0i, r1i), (s0b, s1b)
        gsems, isems = (gs0, gs1), (is0, is1)

        def run(nch_c, row_base):
            ebase = row_base * _CE
            pltpu.sync_copy(col_hbm.at[pl.ds(row_base, nch_c)],
                            cidx.at[pl.ds(0, nch_c)])
            pltpu.sync_copy(row_hbm.at[pl.ds(ebase, _CE)], r0i)
            pltpu.sync_copy(row_hbm.at[pl.ds(ebase + _CE, _CE)], r1i)
            pltpu.sync_copy(s_hbm.at[pl.ds(ebase, _CE)], s0b)
            pltpu.sync_copy(s_hbm.at[pl.ds(ebase + _CE, _CE)], s1b)
            pltpu.sync_copy(m_hbm, m_v)
            plsc.subcore_barrier()
            m16v = m_v[...]

            def gissue(k, slot):
                pltpu.async_copy(h_hbm.at[cidx.at[k]], gbufs[slot], gsems[slot])

            gissue(0, 0)
            gissue(1, 1)

            def step(t, carry):
                for slot in range(2):
                    k = 2 * t + slot
                    pltpu.make_async_copy(h_hbm.at[cidx.at[k]], gbufs[slot],
                                          gsems[slot]).wait()

                    @pl.when(t > 0)
                    def _(slot=slot, k=k):
                        off = ebase + k * _CE
                        pltpu.make_async_copy(row_hbm.at[pl.ds(off, _CE)],
                                              ribufs[slot], isems[slot]).wait()
                        pltpu.make_async_copy(s_hbm.at[pl.ds(off, _CE)],
                                              sbufs[slot], isems[slot]).wait()

                    gb, sb = gbufs[slot], sbufs[slot]

                    def scale(g, c2, gb=gb, sb=sb):
                        sv = sb[pl.ds(g * _L, _L)]
                        w16 = jnp.exp(sv - m16v)
                        for l in range(_L):
                            e = g * _L + l
                            wsc = w16[l]
                            for dd in range(ndl):
                                sl = pl.ds(dd * _L, _L)
                                gb[e, sl] = gb[e, sl] * wsc
                        return c2

                    lax.fori_loop(0, _CE // _L, scale, 0)
                    pltpu.sync_copy(gb, acc.at[ribufs[slot]], add=True)

                    @pl.when(k + 2 < nch_c)
                    def _(k=k, slot=slot):
                        off2 = ebase + (k + 2) * _CE
                        pltpu.async_copy(row_hbm.at[pl.ds(off2, _CE)],
                                         ribufs[slot], isems[slot])
                        pltpu.async_copy(s_hbm.at[pl.ds(off2, _CE)],
                                         sbufs[slot], isems[slot])
                        gissue(k + 2, slot)
                return carry

            lax.fori_loop(0, nch_c // 2, step, 0)

        @pl.when(cid == _BIGCORE)
        def _():
            run(nf, sid * nf)

        @pl.when(cid != _BIGCORE)
        def _():
            run(ns, _NS * nf + sid * ns)

        plsc.subcore_barrier()

        for j in range(ndrain):
            r0 = sid * rps + j * drain
            pltpu.sync_copy(acc.at[pl.ds(r0, drain)], g0)
            pltpu.sync_copy(g0, out_hbm.at[cid, pl.ds(r0, drain)])

    part = _sc_scatter(h, col2d, row1d, s1d, m16)

    out = pl.pallas_call(
        _final_body,
        grid=(nb,),
        in_specs=[pl.BlockSpec((_NC, BR, D), lambda i: (0, i, 0)),
                  pl.BlockSpec((1, D), lambda i: (0, 0))],
        out_specs=pl.BlockSpec((BR, D), lambda i: (i, 0)),
        out_shape=jax.ShapeDtypeStruct((N, D), jnp.float32),
    )(part, izvec)
    return out


# speedup vs baseline: 1.2081x; 1.0218x over previous
import functools

import jax
import jax.numpy as jnp
from jax import lax
from jax.experimental import pallas as pl
from jax.experimental.pallas import tpu as pltpu
from jax.experimental.pallas import tpu_sc as plsc

_NC, _NS, _L = 2, 16, 16
_NW = _NC * _NS
_CE = 128
_FBIG = 80
_BIGCORE = 1


def _proj_body(h_ref, wa_ref, wb_ref, ha_ref, hb_ref):
    hblk = h_ref[...]
    ha_ref[...] = jnp.dot(hblk, wa_ref[...], preferred_element_type=jnp.float32)
    hb_ref[...] = jnp.dot(hblk, wb_ref[...], preferred_element_type=jnp.float32)


def _score_body(ea_ref, gs_ref, w1c_ref, b1_ref, w2_ref, b2_ref,
                s_ref, m_ref, iz_ref, rm, rz):
    i = pl.program_id(0)
    z = jnp.dot(ea_ref[...], w1c_ref[...], preferred_element_type=jnp.float32)
    z = z + gs_ref[...] + b1_ref[...]
    x = jnp.where(z > 0, z, jnp.exp(jnp.minimum(z, 0.0)) - 1.0)
    s = jnp.sum(x * w2_ref[...], axis=1, keepdims=True) + b2_ref[0, 0]
    s = jnp.where(s >= 0, s, 0.2 * s)
    s_ref[...] = s

    bm = jnp.max(s)
    bsum = jnp.sum(jnp.exp(s - bm))

    @pl.when(i == 0)
    def _():
        rm[...] = jnp.full(rm.shape, bm)
        rz[...] = jnp.full(rz.shape, bsum)

    @pl.when(i > 0)
    def _():
        rm_old = rm[...]
        bm_v = jnp.full(rm.shape, bm)
        nm = jnp.maximum(rm_old, bm_v)
        rz[...] = rz[...] * jnp.exp(rm_old - nm) + jnp.full(rz.shape, bsum) * jnp.exp(bm_v - nm)
        rm[...] = nm

    m_ref[...] = rm[...]
    iz_ref[...] = 1.0 / rz[...]


def _final_body(p_ref, iz_ref, o_ref):
    o_ref[...] = (p_ref[0] + p_ref[1]) * iz_ref[...]


def kernel(h, edge_index, edge_attr, W1, b1, W2, b2):
    N, D = h.shape
    E = edge_index.shape[1]
    ndl = D // _L

    row = edge_index[0]
    col = edge_index[1]
    W1a, W1b, W1c = W1[:D], W1[D:2 * D], W1[2 * D:]

    nch = ((-(-E // (_NW * _CE)) + 7) // 8) * 8
    erows = _NW * nch
    epad = erows * _CE
    padn = epad - E
    row2d = jnp.concatenate([row, jnp.zeros((padn,), jnp.int32)]).reshape(erows, _CE)
    col2d = jnp.concatenate([col, jnp.zeros((padn,), jnp.int32)]).reshape(erows, _CE)

    BR = 1000
    nb = N // BR
    hA, hB = pl.pallas_call(
        _proj_body,
        grid=(nb,),
        in_specs=[pl.BlockSpec((BR, D), lambda i: (i, 0)),
                  pl.BlockSpec((D, D), lambda i: (0, 0)),
                  pl.BlockSpec((D, D), lambda i: (0, 0))],
        out_specs=[pl.BlockSpec((BR, D), lambda i: (i, 0)),
                   pl.BlockSpec((BR, D), lambda i: (i, 0))],
        out_shape=[jax.ShapeDtypeStruct((N, D), jnp.float32),
                   jax.ShapeDtypeStruct((N, D), jnp.float32)],
    )(h, W1a, W1b)

    mesh = plsc.VectorSubcoreMesh(core_axis_name="c", subcore_axis_name="s",
                                  num_cores=_NC, num_subcores=_NS)

    nf, ns = _FBIG, nch * 2 - _FBIG

    @functools.partial(
        pl.kernel,
        out_type=jax.ShapeDtypeStruct((epad, D), jnp.float32),
        mesh=mesh,
        scratch_types=[pltpu.VMEM((nf, _CE), jnp.int32),
                       pltpu.VMEM((nf, _CE), jnp.int32),
                       pltpu.VMEM((_CE, D), jnp.float32),
                       pltpu.VMEM((_CE, D), jnp.float32),
                       pltpu.VMEM((_CE, D), jnp.float32),
                       pltpu.VMEM((_CE, D), jnp.float32),
                       pltpu.VMEM((_CE, D), jnp.float32),
                       pltpu.VMEM((_CE, D), jnp.float32),
                       pltpu.SemaphoreType.DMA,
                       pltpu.SemaphoreType.DMA,
                       pltpu.SemaphoreType.DMA,
                       pltpu.SemaphoreType.DMA,
                       pltpu.SemaphoreType.DMA,
                       pltpu.SemaphoreType.DMA],
    )
    def _sc_gather_sum(ha_hbm, hb_hbm, row_hbm, col_hbm, out_hbm,
                       idxr, idxc, a0, b0, o0, a1, b1s, o1,
                       ga0, gb0, ga1, gb1, w0, w1s):
        cid = lax.axis_index("c")
        sid = lax.axis_index("s")
        abufs, bbufs, obufs = (a0, a1), (b0, b1s), (o0, o1)
        gasems, gbsems, wsems = (ga0, ga1), (gb0, gb1), (w0, w1s)

        def run(nch_c, row_base):
            pltpu.sync_copy(row_hbm.at[pl.ds(row_base, nch_c)],
                            idxr.at[pl.ds(0, nch_c)])
            pltpu.sync_copy(col_hbm.at[pl.ds(row_base, nch_c)],
                            idxc.at[pl.ds(0, nch_c)])

            def gissue(k, slot):
                pltpu.async_copy(ha_hbm.at[idxr.at[k]], abufs[slot], gasems[slot])
                pltpu.async_copy(hb_hbm.at[idxc.at[k]], bbufs[slot], gbsems[slot])

            gissue(0, 0)
            gissue(1, 1)

            def step(t, carry):
                for slot in range(2):
                    k = 2 * t + slot
                    pltpu.make_async_copy(ha_hbm.at[idxr.at[k]], abufs[slot],
                                          gasems[slot]).wait()
                    pltpu.make_async_copy(hb_hbm.at[idxc.at[k]], bbufs[slot],
                                          gbsems[slot]).wait()

                    @pl.when(t > 0)
                    def _(slot=slot):
                        pltpu.make_async_copy(obufs[slot], out_hbm.at[pl.ds(0, _CE)],
                                              wsems[slot]).wait()

                    a, b, o = abufs[slot], bbufs[slot], obufs[slot]

                    def addrow(r, c2, a=a, b=b, o=o):
                        for dd in range(ndl):
                            sl = pl.ds(dd * _L, _L)
                            o[r, sl] = a[r, sl] + b[r, sl]
                        return c2

                    lax.fori_loop(0, _CE, addrow, 0)

                    @pl.when(k + 2 < nch_c)
                    def _(k=k, slot=slot):
                        gissue(k + 2, slot)

                    base = (row_base + k) * _CE
                    pltpu.async_copy(o, out_hbm.at[pl.ds(base, _CE)], wsems[slot])
                return carry

            lax.fori_loop(0, nch_c // 2, step, 0)
            pltpu.make_async_copy(o0, out_hbm.at[pl.ds(0, _CE)], w0).wait()
            pltpu.make_async_copy(o1, out_hbm.at[pl.ds(0, _CE)], w1s).wait()

        @pl.when(cid == _BIGCORE)
        def _():
            run(nf, sid * nf)

        @pl.when(cid != _BIGCORE)
        def _():
            run(ns, _NS * nf + sid * ns)

    gsum = _sc_gather_sum(hA, hB, row2d, col2d)

    EB = 10000
    nbe = E // EB
    b2r = jnp.broadcast_to(b2.reshape(1, 1), (1, D))
    s, mvec, izvec = pl.pallas_call(
        _score_body,
        grid=(nbe,),
        in_specs=[pl.BlockSpec((EB, D), lambda i: (i, 0)),
                  pl.BlockSpec((EB, D), lambda i: (i, 0)),
                  pl.BlockSpec((D, D), lambda i: (0, 0)),
                  pl.BlockSpec((1, D), lambda i: (0, 0)),
                  pl.BlockSpec((1, D), lambda i: (0, 0)),
                  pl.BlockSpec((1, D), lambda i: (0, 0))],
        out_specs=[pl.BlockSpec((EB, 1), lambda i: (i, 0)),
                   pl.BlockSpec((1, D), lambda i: (0, 0)),
                   pl.BlockSpec((1, D), lambda i: (0, 0))],
        out_shape=[jax.ShapeDtypeStruct((E, 1), jnp.float32),
                   jax.ShapeDtypeStruct((1, D), jnp.float32),
                   jax.ShapeDtypeStruct((1, D), jnp.float32)],
        scratch_shapes=[pltpu.VMEM((1, D), jnp.float32),
                        pltpu.VMEM((1, D), jnp.float32)],
    )(edge_attr, gsum, W1c, b1.reshape(1, D), W2.reshape(1, D), b2r)

    s1d = jnp.concatenate([s.reshape(E), jnp.full((padn,), -1e30, jnp.float32)])
    row1d = jnp.concatenate([row, jnp.zeros((padn,), jnp.int32)])
    m16 = lax.slice(mvec, (0, 0), (1, _L)).reshape(_L)

    drain = 128
    npad = ((N + drain * _NS - 1) // (drain * _NS)) * (drain * _NS)
    rps = npad // _NS
    ndrain = rps // drain

    @functools.partial(
        pl.kernel,
        out_type=jax.ShapeDtypeStruct((_NC, npad, D), jnp.float32),
        mesh=mesh,
        scratch_types=[pltpu.VMEM((nf, _CE), jnp.int32),
                       pltpu.VMEM((_CE,), jnp.int32),
                       pltpu.VMEM((_CE,), jnp.int32),
                       pltpu.VMEM((_CE,), jnp.float32),
                       pltpu.VMEM((_CE,), jnp.float32),
                       pltpu.VMEM((_CE, D), jnp.float32),
                       pltpu.VMEM((_CE, D), jnp.float32),
                       pltpu.VMEM((_L,), jnp.float32),
                       pltpu.VMEM_SHARED((npad, D), jnp.float32),
                       pltpu.SemaphoreType.DMA,
                       pltpu.SemaphoreType.DMA,
                       pltpu.SemaphoreType.DMA,
                       pltpu.SemaphoreType.DMA],
    )
    def _sc_scatter(h_hbm, col_hbm, row_hbm, s_hbm, m_hbm, out_hbm,
                    cidx, r0i, r1i, s0b, s1b, g0, g1, m_v, acc,
                    gs0, gs1, is0, is1):
        cid = lax.axis_index("c")
        sid = lax.axis_index("s")

        def zrow(r, c2):
            for dd in range(ndl):
                g0[r, pl.ds(dd * _L, _L)] = jnp.zeros((_L,), jnp.float32)
            return c2

        lax.fori_loop(0, _CE, zrow, 0)
        for j in range(ndrain):
            pltpu.sync_copy(g0, acc.at[pl.ds(sid * rps + j * drain, drain)])

        gbufs = (g0, g1)
        ribufs, sbufs = (r0i, r1i), (s0b, s1b)
        gsems, isems = (gs0, gs1), (is0, is1)

        def run(nch_c, row_base):
            ebase = row_base * _CE
            pltpu.sync_copy(col_hbm.at[pl.ds(row_base, nch_c)],
                            cidx.at[pl.ds(0, nch_c)])
            pltpu.sync_copy(row_hbm.at[pl.ds(ebase, _CE)], r0i)
            pltpu.sync_copy(row_hbm.at[pl.ds(ebase + _CE, _CE)], r1i)
            pltpu.sync_copy(s_hbm.at[pl.ds(ebase, _CE)], s0b)
            pltpu.sync_copy(s_hbm.at[pl.ds(ebase + _CE, _CE)], s1b)
            pltpu.sync_copy(m_hbm, m_v)
            plsc.subcore_barrier()
            m16v = m_v[...]

            def gissue(k, slot):
                pltpu.async_copy(h_hbm.at[cidx.at[k]], gbufs[slot], gsems[slot])

            gissue(0, 0)
            gissue(1, 1)

            def step(t, carry):
                for slot in range(2):
                    k = 2 * t + slot
                    pltpu.make_async_copy(h_hbm.at[cidx.at[k]], gbufs[slot],
                                          gsems[slot]).wait()

                    @pl.when(t > 0)
                    def _(slot=slot, k=k):
                        off = ebase + k * _CE
                        pltpu.make_async_copy(row_hbm.at[pl.ds(off, _CE)],
                                              ribufs[slot], isems[slot]).wait()
                        pltpu.make_async_copy(s_hbm.at[pl.ds(off, _CE)],
                                              sbufs[slot], isems[slot]).wait()

                    gb, sb = gbufs[slot], sbufs[slot]

                    def scale(g, c2, gb=gb, sb=sb):
                        sv = sb[pl.ds(g * _L, _L)]
                        w16 = jnp.exp(sv - m16v)
                        for l in range(_L):
                            e = g * _L + l
                            wsc = w16[l]
                            for dd in range(ndl):
                                sl = pl.ds(dd * _L, _L)
                                gb[e, sl] = gb[e, sl] * wsc
                        return c2

                    lax.fori_loop(0, _CE // _L, scale, 0)
                    pltpu.sync_copy(gb, acc.at[ribufs[slot]], add=True)

                    @pl.when(k + 2 < nch_c)
                    def _(k=k, slot=slot):
                        off2 = ebase + (k + 2) * _CE
                        pltpu.async_copy(row_hbm.at[pl.ds(off2, _CE)],
                                         ribufs[slot], isems[slot])
                        pltpu.async_copy(s_hbm.at[pl.ds(off2, _CE)],
                                         sbufs[slot], isems[slot])
                        gissue(k + 2, slot)
                return carry

            lax.fori_loop(0, nch_c // 2, step, 0)

        @pl.when(cid == _BIGCORE)
        def _():
            run(nf, sid * nf)

        @pl.when(cid != _BIGCORE)
        def _():
            run(ns, _NS * nf + sid * ns)

        plsc.subcore_barrier()

        for j in range(ndrain):
            r0 = sid * rps + j * drain
            pltpu.sync_copy(acc.at[pl.ds(r0, drain)], g0)
            pltpu.sync_copy(g0, out_hbm.at[cid, pl.ds(r0, drain)])

    part = _sc_scatter(h, col2d, row1d, s1d, m16)

    out = pl.pallas_call(
        _final_body,
        grid=(nb,),
        in_specs=[pl.BlockSpec((_NC, BR, D), lambda i: (0, i, 0)),
                  pl.BlockSpec((1, D), lambda i: (0, 0))],
        out_specs=pl.BlockSpec((BR, D), lambda i: (i, 0)),
        out_shape=jax.ShapeDtypeStruct((N, D), jnp.float32),
    )(part, izvec)
    return out
